# Initial kernel scaffold; baseline (speedup 1.0000x reference)
import jax
import jax.numpy as jnp
from jax.experimental import pallas as pl


def kernel(x, edge_index, W1, b1, W2, b2):
    raise NotImplementedError("write your pallas kernel here")


# speedup vs baseline: 31.0575x; 31.0575x over previous
import functools

import jax
import jax.numpy as jnp
from jax import lax
from jax.experimental import pallas as pl
from jax.experimental.pallas import tpu as pltpu
from jax.experimental.pallas import tpu_sc as plsc

N = 10000
NPAD = 10240
F_IN = 128
HID = 16
FW = 16
F_OUT = 8

NC = 2
NS = 16
NW = NC * NS
CB = 128
RPT = NPAD // NS
DUMMY = NPAD - 1

_mesh = plsc.VectorSubcoreMesh(
    core_axis_name="c", subcore_axis_name="s", num_cores=NC, num_subcores=NS
)
_sc_params = pltpu.CompilerParams(use_tc_tiling_on_sc=False)


def _num_chunks(total_edges):
    return -(-total_edges // (NW * CB))


def _make_deg(ch):
    @functools.partial(
        pl.kernel,
        mesh=_mesh,
        out_type=jax.ShapeDtypeStruct((NC, NPAD), jnp.float32),
        scratch_types=[
            pltpu.VMEM((ch, CB), jnp.int32),
            pltpu.VMEM((CB,), jnp.float32),
            pltpu.VMEM((RPT,), jnp.float32),
            pltpu.VMEM_SHARED((NPAD,), jnp.float32),
        ],
        compiler_params=_sc_params,
    )
    def deg_kernel(dst_hbm, out_hbm, dst_v, ones_v, buf_v, acc):
        c = lax.axis_index("c")
        s = lax.axis_index("s")
        wid = s * NC + c
        for i in range(CB // 16):
            ones_v[pl.ds(i * 16, 16)] = jnp.ones((16,), jnp.float32)

        def zb(r, carry):
            buf_v[pl.ds(r * 16, 16)] = jnp.zeros((16,), jnp.float32)
            return carry

        lax.fori_loop(0, RPT // 16, zb, 0)
        pltpu.sync_copy(buf_v, acc.at[pl.ds(s * RPT, RPT)])
        pltpu.sync_copy(dst_hbm.at[wid], dst_v)
        plsc.subcore_barrier()

        def body(j, carry):
            pltpu.sync_copy(ones_v, acc.at[dst_v.at[j]], add=True)
            return carry

        lax.fori_loop(0, ch, body, 0)
        plsc.subcore_barrier()
        pltpu.sync_copy(acc.at[pl.ds(s * RPT, RPT)], buf_v)
        pltpu.sync_copy(buf_v, out_hbm.at[c, pl.ds(s * RPT, RPT)])

    return deg_kernel


def _make_agg(ch):
    @functools.partial(
        pl.kernel,
        mesh=_mesh,
        out_type=jax.ShapeDtypeStruct((NC, NPAD, FW), jnp.float32),
        scratch_types=[
            pltpu.VMEM((ch, CB), jnp.int32),
            pltpu.VMEM((ch, CB), jnp.int32),
            pltpu.VMEM((CB, FW), jnp.float32),
            pltpu.VMEM((RPT, FW), jnp.float32),
            pltpu.VMEM_SHARED((NPAD, FW), jnp.float32),
            pltpu.SemaphoreType.DMA,
        ],
        compiler_params=_sc_params,
    )
    def agg_kernel(tbl_hbm, src_hbm, dst_hbm, out_hbm,
                   src_v, dst_v, rows_v, buf_v, acc, sem):
        c = lax.axis_index("c")
        s = lax.axis_index("s")
        wid = s * NC + c

        def zb(r, carry):
            buf_v[r] = jnp.zeros((FW,), jnp.float32)
            return carry

        lax.fori_loop(0, RPT, zb, 0)
        pltpu.sync_copy(buf_v, acc.at[pl.ds(s * RPT, RPT)])
        pltpu.sync_copy(src_hbm.at[wid], src_v)
        pltpu.sync_copy(dst_hbm.at[wid], dst_v)
        plsc.subcore_barrier()

        def body(j, carry):
            pltpu.async_copy(tbl_hbm.at[src_v.at[j]], rows_v, sem).wait()
            pltpu.sync_copy(rows_v, acc.at[dst_v.at[j]], add=True)
            return carry

        lax.fori_loop(0, ch, body, 0)
        plsc.subcore_barrier()
        pltpu.sync_copy(acc.at[pl.ds(s * RPT, RPT)], buf_v)
        pltpu.sync_copy(buf_v, out_hbm.at[c, pl.ds(s * RPT, RPT)])

    return agg_kernel


_BR = 512


def _k1_body(x_ref, w_ref, degp_ref, h_ref, dis_ref):
    deg = jnp.maximum(degp_ref[0] + degp_ref[1], 1.0)
    dis = lax.rsqrt(deg)
    h = jnp.dot(x_ref[...], w_ref[...], preferred_element_type=jnp.float32)
    h_ref[...] = h * dis
    dis_ref[...] = dis


def _k1(x_pad, w1, degp):
    grid = (NPAD // _BR,)
    return pl.pallas_call(
        _k1_body,
        grid=grid,
        in_specs=[
            pl.BlockSpec((_BR, F_IN), lambda i: (i, 0)),
            pl.BlockSpec((F_IN, HID), lambda i: (0, 0)),
            pl.BlockSpec((NC, _BR, 1), lambda i: (0, i, 0)),
        ],
        out_specs=[
            pl.BlockSpec((_BR, HID), lambda i: (i, 0)),
            pl.BlockSpec((_BR, 1), lambda i: (i, 0)),
        ],
        out_shape=[
            jax.ShapeDtypeStruct((NPAD, HID), jnp.float32),
            jax.ShapeDtypeStruct((NPAD, 1), jnp.float32),
        ],
    )(x_pad, w1, degp)


def _k3_body(a_ref, dis_ref, w_ref, b_ref, h_ref):
    agg = a_ref[0] + a_ref[1]
    dis = dis_ref[...]
    z = jnp.maximum(dis * agg + b_ref[...], 0.0)
    h_ref[...] = dis * jnp.dot(z, w_ref[...], preferred_element_type=jnp.float32)


def _k3(accp, dis, w2p, b1):
    grid = (NPAD // _BR,)
    return pl.pallas_call(
        _k3_body,
        grid=grid,
        in_specs=[
            pl.BlockSpec((NC, _BR, FW), lambda i: (0, i, 0)),
            pl.BlockSpec((_BR, 1), lambda i: (i, 0)),
            pl.BlockSpec((HID, FW), lambda i: (0, 0)),
            pl.BlockSpec((1, HID), lambda i: (0, 0)),
        ],
        out_specs=pl.BlockSpec((_BR, FW), lambda i: (i, 0)),
        out_shape=jax.ShapeDtypeStruct((NPAD, FW), jnp.float32),
    )(accp, dis, w2p, b1)


def _k5_body(a_ref, dis_ref, b_ref, o_ref):
    agg = a_ref[0] + a_ref[1]
    o_ref[...] = dis_ref[...] * agg[:, :F_OUT] + b_ref[...]


def _k5(accp, dis, b2):
    grid = (NPAD // _BR,)
    return pl.pallas_call(
        _k5_body,
        grid=grid,
        in_specs=[
            pl.BlockSpec((NC, _BR, FW), lambda i: (0, i, 0)),
            pl.BlockSpec((_BR, 1), lambda i: (i, 0)),
            pl.BlockSpec((1, F_OUT), lambda i: (0, 0)),
        ],
        out_specs=pl.BlockSpec((_BR, F_OUT), lambda i: (i, 0)),
        out_shape=jax.ShapeDtypeStruct((NPAD, F_OUT), jnp.float32),
    )(accp, dis, b2)


def kernel(x, edge_index, W1, b1, W2, b2):
    e = edge_index.shape[1]
    total = e + N
    ch = _num_chunks(total)
    padded = NW * ch * CB
    loop = jnp.arange(N, dtype=jnp.int32)
    src_all = jnp.concatenate(
        [edge_index[0], loop, jnp.zeros((padded - total,), jnp.int32)]
    ).reshape(NW, ch, CB)
    dst_all = jnp.concatenate(
        [edge_index[1], loop, jnp.full((padded - total,), DUMMY, jnp.int32)]
    ).reshape(NW, ch, CB)

    degp = _make_deg(ch)(dst_all)
    x_pad = jnp.pad(x, ((0, NPAD - N), (0, 0)))
    h1p, dis = _k1(x_pad, W1, degp.reshape(NC, NPAD, 1))

    agg = _make_agg(ch)
    acc1p = agg(h1p, src_all, dst_all)
    w2p = jnp.pad(W2, ((0, 0), (0, FW - F_OUT)))
    h2p = _k3(acc1p, dis, w2p, b1.reshape(1, HID))
    acc2p = agg(h2p, src_all, dst_all)
    outp = _k5(acc2p, dis, b2.reshape(1, F_OUT))
    return outp[:N]


# speedup vs baseline: 44.6672x; 1.4382x over previous
import functools

import jax
import jax.numpy as jnp
from jax import lax
from jax.experimental import pallas as pl
from jax.experimental.pallas import tpu as pltpu
from jax.experimental.pallas import tpu_sc as plsc

N = 10000
NPAD = 10240
F_IN = 128
HID = 16
FW = 16
F_OUT = 8

NC = 2
NS = 16
NW = NC * NS
CB = 128
RPT = NPAD // NS
DUMMY = NPAD - 1

_mesh = plsc.VectorSubcoreMesh(
    core_axis_name="c", subcore_axis_name="s", num_cores=NC, num_subcores=NS
)
_sc_params = pltpu.CompilerParams(use_tc_tiling_on_sc=False)


NB = 27


def _num_chunks(total_edges):
    ch = -(-total_edges // (NW * CB))
    return -(-ch // NB) * NB


def _make_deg(ch):
    @functools.partial(
        pl.kernel,
        mesh=_mesh,
        out_type=jax.ShapeDtypeStruct((NC, NPAD), jnp.float32),
        scratch_types=[
            pltpu.VMEM((ch, CB), jnp.int32),
            pltpu.VMEM((CB,), jnp.float32),
            pltpu.VMEM((RPT,), jnp.float32),
            pltpu.VMEM_SHARED((NPAD,), jnp.float32),
            pltpu.SemaphoreType.DMA,
        ],
        compiler_params=_sc_params,
    )
    def deg_kernel(dst_hbm, out_hbm, dst_v, ones_v, buf_v, acc, sem):
        c = lax.axis_index("c")
        s = lax.axis_index("s")
        wid = s * NC + c
        for i in range(CB // 16):
            ones_v[pl.ds(i * 16, 16)] = jnp.ones((16,), jnp.float32)

        def zb(r, carry):
            buf_v[pl.ds(r * 16, 16)] = jnp.zeros((16,), jnp.float32)
            return carry

        lax.fori_loop(0, RPT // 16, zb, 0)
        pltpu.sync_copy(buf_v, acc.at[pl.ds(s * RPT, RPT)])
        pltpu.sync_copy(dst_hbm.at[wid], dst_v)
        plsc.subcore_barrier()

        def body(r, carry):
            descs = [
                pltpu.async_copy(ones_v, acc.at[dst_v.at[r * NB + b]], sem,
                                 add=True)
                for b in range(NB)
            ]
            for d in descs:
                d.wait()
            return carry

        lax.fori_loop(0, ch // NB, body, 0)
        plsc.subcore_barrier()
        pltpu.sync_copy(acc.at[pl.ds(s * RPT, RPT)], buf_v)
        pltpu.sync_copy(buf_v, out_hbm.at[c, pl.ds(s * RPT, RPT)])

    return deg_kernel


def _make_agg(ch):
    @functools.partial(
        pl.kernel,
        mesh=_mesh,
        out_type=jax.ShapeDtypeStruct((NC, NPAD, FW), jnp.float32),
        scratch_types=[
            pltpu.VMEM((ch, CB), jnp.int32),
            pltpu.VMEM((ch, CB), jnp.int32),
            pltpu.VMEM((NB, CB, FW), jnp.float32),
            pltpu.VMEM((RPT, FW), jnp.float32),
            pltpu.VMEM_SHARED((NPAD, FW), jnp.float32),
            pltpu.SemaphoreType.DMA,
            pltpu.SemaphoreType.DMA,
        ],
        compiler_params=_sc_params,
    )
    def agg_kernel(tbl_hbm, src_hbm, dst_hbm, out_hbm,
                   src_v, dst_v, rows_v, buf_v, acc, gsem, ssem):
        c = lax.axis_index("c")
        s = lax.axis_index("s")
        wid = s * NC + c

        def zb(r, carry):
            buf_v[r] = jnp.zeros((FW,), jnp.float32)
            return carry

        lax.fori_loop(0, RPT, zb, 0)
        pltpu.sync_copy(buf_v, acc.at[pl.ds(s * RPT, RPT)])
        pltpu.sync_copy(src_hbm.at[wid], src_v)
        pltpu.sync_copy(dst_hbm.at[wid], dst_v)
        plsc.subcore_barrier()

        def body(r, carry):
            j0 = r * NB
            gds = [
                pltpu.async_copy(tbl_hbm.at[src_v.at[j0 + b]], rows_v.at[b],
                                 gsem)
                for b in range(NB)
            ]
            sds = []
            for b in range(NB):
                gds[b].wait()
                sds.append(
                    pltpu.async_copy(rows_v.at[b], acc.at[dst_v.at[j0 + b]],
                                     ssem, add=True))
            for d in sds:
                d.wait()
            return carry

        lax.fori_loop(0, ch // NB, body, 0)
        plsc.subcore_barrier()
        pltpu.sync_copy(acc.at[pl.ds(s * RPT, RPT)], buf_v)
        pltpu.sync_copy(buf_v, out_hbm.at[c, pl.ds(s * RPT, RPT)])

    return agg_kernel


_BR = 512


def _k1_body(x_ref, w_ref, degp_ref, h_ref, dis_ref):
    deg = jnp.maximum(degp_ref[0] + degp_ref[1], 1.0)
    dis = lax.rsqrt(deg)
    h = jnp.dot(x_ref[...], w_ref[...], preferred_element_type=jnp.float32)
    h_ref[...] = h * dis
    dis_ref[...] = dis


def _k1(x_pad, w1, degp):
    grid = (NPAD // _BR,)
    return pl.pallas_call(
        _k1_body,
        grid=grid,
        in_specs=[
            pl.BlockSpec((_BR, F_IN), lambda i: (i, 0)),
            pl.BlockSpec((F_IN, HID), lambda i: (0, 0)),
            pl.BlockSpec((NC, _BR, 1), lambda i: (0, i, 0)),
        ],
        out_specs=[
            pl.BlockSpec((_BR, HID), lambda i: (i, 0)),
            pl.BlockSpec((_BR, 1), lambda i: (i, 0)),
        ],
        out_shape=[
            jax.ShapeDtypeStruct((NPAD, HID), jnp.float32),
            jax.ShapeDtypeStruct((NPAD, 1), jnp.float32),
        ],
    )(x_pad, w1, degp)


def _k3_body(a_ref, dis_ref, w_ref, b_ref, h_ref):
    agg = a_ref[0] + a_ref[1]
    dis = dis_ref[...]
    z = jnp.maximum(dis * agg + b_ref[...], 0.0)
    h_ref[...] = dis * jnp.dot(z, w_ref[...], preferred_element_type=jnp.float32)


def _k3(accp, dis, w2p, b1):
    grid = (NPAD // _BR,)
    return pl.pallas_call(
        _k3_body,
        grid=grid,
        in_specs=[
            pl.BlockSpec((NC, _BR, FW), lambda i: (0, i, 0)),
            pl.BlockSpec((_BR, 1), lambda i: (i, 0)),
            pl.BlockSpec((HID, FW), lambda i: (0, 0)),
            pl.BlockSpec((1, HID), lambda i: (0, 0)),
        ],
        out_specs=pl.BlockSpec((_BR, FW), lambda i: (i, 0)),
        out_shape=jax.ShapeDtypeStruct((NPAD, FW), jnp.float32),
    )(accp, dis, w2p, b1)


def _k5_body(a_ref, dis_ref, b_ref, o_ref):
    agg = a_ref[0] + a_ref[1]
    o_ref[...] = dis_ref[...] * agg[:, :F_OUT] + b_ref[...]


def _k5(accp, dis, b2):
    grid = (NPAD // _BR,)
    return pl.pallas_call(
        _k5_body,
        grid=grid,
        in_specs=[
            pl.BlockSpec((NC, _BR, FW), lambda i: (0, i, 0)),
            pl.BlockSpec((_BR, 1), lambda i: (i, 0)),
            pl.BlockSpec((1, F_OUT), lambda i: (0, 0)),
        ],
        out_specs=pl.BlockSpec((_BR, F_OUT), lambda i: (i, 0)),
        out_shape=jax.ShapeDtypeStruct((NPAD, F_OUT), jnp.float32),
    )(accp, dis, b2)


def kernel(x, edge_index, W1, b1, W2, b2):
    e = edge_index.shape[1]
    total = e + N
    ch = _num_chunks(total)
    padded = NW * ch * CB
    loop = jnp.arange(N, dtype=jnp.int32)
    src_all = jnp.concatenate(
        [edge_index[0], loop, jnp.zeros((padded - total,), jnp.int32)]
    ).reshape(NW, ch, CB)
    dst_all = jnp.concatenate(
        [edge_index[1], loop, jnp.full((padded - total,), DUMMY, jnp.int32)]
    ).reshape(NW, ch, CB)

    degp = _make_deg(ch)(dst_all)
    x_pad = jnp.pad(x, ((0, NPAD - N), (0, 0)))
    h1p, dis = _k1(x_pad, W1, degp.reshape(NC, NPAD, 1))

    agg = _make_agg(ch)
    acc1p = agg(h1p, src_all, dst_all)
    w2p = jnp.pad(W2, ((0, 0), (0, FW - F_OUT)))
    h2p = _k3(acc1p, dis, w2p, b1.reshape(1, HID))
    acc2p = agg(h2p, src_all, dst_all)
    outp = _k5(acc2p, dis, b2.reshape(1, F_OUT))
    return outp[:N]


# speedup vs baseline: 76.0956x; 1.7036x over previous
import functools

import jax
import jax.numpy as jnp
from jax import lax
from jax.experimental import pallas as pl
from jax.experimental.pallas import tpu as pltpu
from jax.experimental.pallas import tpu_sc as plsc

N = 10000
NPAD = 10240
F_IN = 128
HID = 16
FW = 16
F_OUT = 8
PR = NPAD * FW // 128

NC = 2
NS = 16
NW = NC * NS
CB = 125
CH = 80
NB = 20
RPT = NPAD // NS

_mesh = plsc.VectorSubcoreMesh(
    core_axis_name="c", subcore_axis_name="s", num_cores=NC, num_subcores=NS
)
_sc_params = pltpu.CompilerParams(use_tc_tiling_on_sc=False)


@functools.partial(
    pl.kernel,
    mesh=_mesh,
    out_type=jax.ShapeDtypeStruct((NC, NPAD, FW), jnp.float32),
    scratch_types=[
        pltpu.VMEM((CH, CB), jnp.int32),
        pltpu.VMEM((128,), jnp.float32),
        pltpu.VMEM((RPT,), jnp.float32),
        pltpu.VMEM((RPT, FW), jnp.float32),
        pltpu.VMEM_SHARED((NPAD,), jnp.float32),
        pltpu.SemaphoreType.DMA,
    ],
    compiler_params=_sc_params,
)
def _deg(dst_hbm, out_hbm, dst_v, ones_v, st_v, buf_v, acc, sem):
    c = lax.axis_index("c")
    s = lax.axis_index("s")
    wid = s * NC + c
    for i in range(8):
        ones_v[pl.ds(i * 16, 16)] = jnp.ones((16,), jnp.float32)
    for i in range(RPT // 16):
        st_v[pl.ds(i * 16, 16)] = jnp.zeros((16,), jnp.float32)
    pltpu.sync_copy(st_v, acc.at[pl.ds(s * RPT, RPT)])
    pltpu.sync_copy(dst_hbm.at[wid], dst_v)
    plsc.subcore_barrier()

    ones_row = ones_v.at[pl.ds(0, CB)]

    def body(r, carry):
        descs = [
            pltpu.async_copy(ones_row, acc.at[dst_v.at[r * NB + b]], sem,
                             add=True)
            for b in range(NB)
        ]
        for d in descs:
            d.wait()
        return carry

    lax.fori_loop(0, CH // NB, body, 0)
    plsc.subcore_barrier()
    pltpu.sync_copy(acc.at[pl.ds(s * RPT, RPT)], st_v)

    def rep(g, carry):
        v = st_v[pl.ds(g * 16, 16)]
        for l in range(16):
            buf_v[g * 16 + l] = jnp.broadcast_to(v[l], (FW,))
        return carry

    lax.fori_loop(0, RPT // 16, rep, 0)
    pltpu.sync_copy(buf_v, out_hbm.at[c, pl.ds(s * RPT, RPT)])


@functools.partial(
    pl.kernel,
    mesh=_mesh,
    out_type=jax.ShapeDtypeStruct((NC, NPAD, FW), jnp.float32),
    scratch_types=[
        pltpu.VMEM((CH, CB), jnp.int32),
        pltpu.VMEM((CH, CB), jnp.int32),
        pltpu.VMEM((NB, CB, FW), jnp.float32),
        pltpu.VMEM((RPT, FW), jnp.float32),
        pltpu.VMEM_SHARED((NPAD, FW), jnp.float32),
        pltpu.SemaphoreType.DMA,
        pltpu.SemaphoreType.DMA,
    ],
    compiler_params=_sc_params,
)
def _agg(tbl_hbm, src_hbm, dst_hbm, out_hbm,
         src_v, dst_v, rows_v, buf_v, acc, gsem, ssem):
    c = lax.axis_index("c")
    s = lax.axis_index("s")
    wid = s * NC + c

    def zb(r, carry):
        buf_v[r] = jnp.zeros((FW,), jnp.float32)
        return carry

    lax.fori_loop(0, RPT, zb, 0)
    pltpu.sync_copy(buf_v, acc.at[pl.ds(s * RPT, RPT)])
    pltpu.sync_copy(src_hbm.at[wid], src_v)
    pltpu.sync_copy(dst_hbm.at[wid], dst_v)
    plsc.subcore_barrier()

    def body(r, carry):
        j0 = r * NB
        gds = [
            pltpu.async_copy(tbl_hbm.at[src_v.at[j0 + b]], rows_v.at[b], gsem)
            for b in range(NB)
        ]
        sds = []
        for b in range(NB):
            gds[b].wait()
            sds.append(
                pltpu.async_copy(rows_v.at[b], acc.at[dst_v.at[j0 + b]],
                                 ssem, add=True))
        for d in sds:
            d.wait()
        return carry

    lax.fori_loop(0, CH // NB, body, 0)
    plsc.subcore_barrier()
    pltpu.sync_copy(acc.at[pl.ds(s * RPT, RPT)], buf_v)
    pltpu.sync_copy(buf_v, out_hbm.at[c, pl.ds(s * RPT, RPT)])


_BP = 128


def _k1_body(x_ref, w_ref, degp_ref, h_ref, dis_ref):
    deg = degp_ref[0] + degp_ref[1] + 1.0
    dis = lax.rsqrt(deg)
    h = jnp.dot(x_ref[...], w_ref[...], preferred_element_type=jnp.float32)
    h_ref[...] = h * dis
    dis_ref[...] = dis


def _k1(x_r, w_big, degp):
    return pl.pallas_call(
        _k1_body,
        grid=(PR // _BP,),
        in_specs=[
            pl.BlockSpec((_BP, F_IN * 8), lambda i: (i, 0)),
            pl.BlockSpec((F_IN * 8, 128), lambda i: (0, 0)),
            pl.BlockSpec((NC, _BP, 128), lambda i: (0, i, 0)),
        ],
        out_specs=[
            pl.BlockSpec((_BP, 128), lambda i: (i, 0)),
            pl.BlockSpec((_BP, 128), lambda i: (i, 0)),
        ],
        out_shape=[
            jax.ShapeDtypeStruct((PR, 128), jnp.float32),
            jax.ShapeDtypeStruct((PR, 128), jnp.float32),
        ],
    )(x_r, w_big, degp)


def _k3_body(a_ref, h1_ref, dis_ref, w_ref, b_ref, h_ref):
    dis = dis_ref[...]
    agg = a_ref[0] + a_ref[1] + h1_ref[...]
    z = jnp.maximum(dis * agg + b_ref[...], 0.0)
    h_ref[...] = dis * jnp.dot(z, w_ref[...],
                               preferred_element_type=jnp.float32)


def _k3(accp, h1p, disp, w2_big, b1rep):
    return pl.pallas_call(
        _k3_body,
        grid=(PR // _BP,),
        in_specs=[
            pl.BlockSpec((NC, _BP, 128), lambda i: (0, i, 0)),
            pl.BlockSpec((_BP, 128), lambda i: (i, 0)),
            pl.BlockSpec((_BP, 128), lambda i: (i, 0)),
            pl.BlockSpec((128, 128), lambda i: (0, 0)),
            pl.BlockSpec((1, 128), lambda i: (0, 0)),
        ],
        out_specs=pl.BlockSpec((_BP, 128), lambda i: (i, 0)),
        out_shape=jax.ShapeDtypeStruct((PR, 128), jnp.float32),
    )(accp, h1p, disp, w2_big, b1rep)


def _k5_body(a_ref, h2_ref, dis_ref, b_ref, o_ref):
    agg = a_ref[0] + a_ref[1] + h2_ref[...]
    o_ref[...] = dis_ref[...] * agg + b_ref[...]


def _k5(accp, h2p, disp, b2rep):
    return pl.pallas_call(
        _k5_body,
        grid=(PR // _BP,),
        in_specs=[
            pl.BlockSpec((NC, _BP, 128), lambda i: (0, i, 0)),
            pl.BlockSpec((_BP, 128), lambda i: (i, 0)),
            pl.BlockSpec((_BP, 128), lambda i: (i, 0)),
            pl.BlockSpec((1, 128), lambda i: (0, 0)),
        ],
        out_specs=pl.BlockSpec((_BP, 128), lambda i: (i, 0)),
        out_shape=jax.ShapeDtypeStruct((PR, 128), jnp.float32),
    )(accp, h2p, disp, b2rep)


def kernel(x, edge_index, W1, b1, W2, b2):
    src3 = edge_index[0].reshape(NW, CH, CB)
    dst3 = edge_index[1].reshape(NW, CH, CB)
    eye8 = jnp.eye(8, dtype=jnp.float32)
    w1_big = jnp.kron(eye8, W1)
    w2p = jnp.pad(W2, ((0, 0), (0, FW - F_OUT)))
    w2_big = jnp.kron(eye8, w2p)
    b1rep = jnp.tile(b1, 8).reshape(1, 128)
    b2rep = jnp.tile(jnp.pad(b2, (0, FW - F_OUT)), 8).reshape(1, 128)
    x_r = jnp.pad(x, ((0, NPAD - N), (0, 0))).reshape(PR, F_IN * 8)

    degp = _deg(dst3).reshape(NC, PR, 128)
    h1p, disp = _k1(x_r, w1_big, degp)

    acc1p = _agg(h1p.reshape(NPAD, FW), src3, dst3).reshape(NC, PR, 128)
    h2p = _k3(acc1p, h1p, disp, w2_big, b1rep)
    acc2p = _agg(h2p.reshape(NPAD, FW), src3, dst3).reshape(NC, PR, 128)
    outp = _k5(acc2p, h2p, disp, b2rep)
    return outp.reshape(NPAD, FW)[:N, :F_OUT]


# speedup vs baseline: 89.2537x; 1.1729x over previous
import functools

import jax
import jax.numpy as jnp
from jax import lax
from jax.experimental import pallas as pl
from jax.experimental.pallas import tpu as pltpu
from jax.experimental.pallas import tpu_sc as plsc

N = 10000
NPAD = 10240
F_IN = 128
HID = 16
FW = 16
F_OUT = 8
PR = NPAD * FW // 128

NC = 2
NS = 16
NW = NC * NS
CB = 125
CH = 80
NB = 40
RPT = NPAD // NS

_mesh = plsc.VectorSubcoreMesh(
    core_axis_name="c", subcore_axis_name="s", num_cores=NC, num_subcores=NS
)
_sc_params = pltpu.CompilerParams(use_tc_tiling_on_sc=False)


@functools.partial(
    pl.kernel,
    mesh=_mesh,
    out_type=jax.ShapeDtypeStruct((NC, NPAD, FW), jnp.float32),
    scratch_types=[
        pltpu.VMEM((CH, CB), jnp.int32),
        pltpu.VMEM((128,), jnp.float32),
        pltpu.VMEM((RPT,), jnp.float32),
        pltpu.VMEM((RPT, FW), jnp.float32),
        pltpu.VMEM_SHARED((NPAD,), jnp.float32),
        pltpu.SemaphoreType.DMA,
    ],
    compiler_params=_sc_params,
)
def _deg(ei_hbm, out_hbm, dst_v, ones_v, st_v, buf_v, acc, sem):
    c = lax.axis_index("c")
    s = lax.axis_index("s")
    wid = s * NC + c
    dslab = pltpu.async_copy(ei_hbm.at[1, wid], dst_v, sem)
    for i in range(8):
        ones_v[pl.ds(i * 16, 16)] = jnp.ones((16,), jnp.float32)
    for i in range(RPT // 16):
        st_v[pl.ds(i * 16, 16)] = jnp.zeros((16,), jnp.float32)
    pltpu.sync_copy(st_v, acc.at[pl.ds(s * RPT, RPT)])
    dslab.wait()
    plsc.subcore_barrier()

    ones_row = ones_v.at[pl.ds(0, CB)]

    def body(r, carry):
        descs = [
            pltpu.async_copy(ones_row, acc.at[dst_v.at[r * NB + b]], sem,
                             add=True)
            for b in range(NB)
        ]
        for d in descs:
            d.wait()
        return carry

    lax.fori_loop(0, CH // NB, body, 0)
    plsc.subcore_barrier()
    pltpu.sync_copy(acc.at[pl.ds(s * RPT, RPT)], st_v)

    def rep(g, carry):
        v = st_v[pl.ds(g * 16, 16)]
        for l in range(16):
            buf_v[g * 16 + l] = jnp.broadcast_to(v[l], (FW,))
        return carry

    lax.fori_loop(0, RPT // 16, rep, 0)
    pltpu.sync_copy(buf_v, out_hbm.at[c, pl.ds(s * RPT, RPT)])


@functools.partial(
    pl.kernel,
    mesh=_mesh,
    out_type=jax.ShapeDtypeStruct((NC, NPAD, FW), jnp.float32),
    scratch_types=[
        pltpu.VMEM((CH, CB), jnp.int32),
        pltpu.VMEM((CH, CB), jnp.int32),
        pltpu.VMEM((NB, CB, FW), jnp.float32),
        pltpu.VMEM((RPT, FW), jnp.float32),
        pltpu.VMEM_SHARED((NPAD, FW), jnp.float32),
        pltpu.SemaphoreType.DMA,
        pltpu.SemaphoreType.DMA,
    ],
    compiler_params=_sc_params,
)
def _agg(tbl_hbm, ei_hbm, out_hbm,
         src_v, dst_v, rows_v, buf_v, acc, gsem, ssem):
    c = lax.axis_index("c")
    s = lax.axis_index("s")
    wid = s * NC + c
    sslab = pltpu.async_copy(ei_hbm.at[0, wid], src_v, gsem)
    dslab = pltpu.async_copy(ei_hbm.at[1, wid], dst_v, gsem)

    def zb(r, carry):
        buf_v[r] = jnp.zeros((FW,), jnp.float32)
        return carry

    lax.fori_loop(0, RPT, zb, 0)
    pltpu.sync_copy(buf_v, acc.at[pl.ds(s * RPT, RPT)])
    sslab.wait()
    dslab.wait()
    plsc.subcore_barrier()

    def body(r, carry):
        j0 = r * NB
        gds = [
            pltpu.async_copy(tbl_hbm.at[src_v.at[j0 + b]], rows_v.at[b], gsem)
            for b in range(NB)
        ]
        sds = []
        for b in range(NB):
            gds[b].wait()
            sds.append(
                pltpu.async_copy(rows_v.at[b], acc.at[dst_v.at[j0 + b]],
                                 ssem, add=True))
        for d in sds:
            d.wait()
        return carry

    lax.fori_loop(0, CH // NB, body, 0)
    plsc.subcore_barrier()
    pltpu.sync_copy(acc.at[pl.ds(s * RPT, RPT)], buf_v)
    pltpu.sync_copy(buf_v, out_hbm.at[c, pl.ds(s * RPT, RPT)])


_BP = 128


def _k1_body(x_ref, w_ref, degp_ref, h_ref, dis_ref):
    deg = degp_ref[0] + degp_ref[1] + 1.0
    dis = lax.rsqrt(deg)
    h = jnp.dot(x_ref[...], w_ref[...], preferred_element_type=jnp.float32)
    h_ref[...] = h * dis
    dis_ref[...] = dis


def _k1(x_r, w_big, degp):
    return pl.pallas_call(
        _k1_body,
        grid=(PR // _BP,),
        in_specs=[
            pl.BlockSpec((_BP, F_IN * 8), lambda i: (i, 0)),
            pl.BlockSpec((F_IN * 8, 128), lambda i: (0, 0)),
            pl.BlockSpec((NC, _BP, 128), lambda i: (0, i, 0)),
        ],
        out_specs=[
            pl.BlockSpec((_BP, 128), lambda i: (i, 0)),
            pl.BlockSpec((_BP, 128), lambda i: (i, 0)),
        ],
        out_shape=[
            jax.ShapeDtypeStruct((PR, 128), jnp.float32),
            jax.ShapeDtypeStruct((PR, 128), jnp.float32),
        ],
    )(x_r, w_big, degp)


def _k3_body(a_ref, h1_ref, dis_ref, w_ref, b_ref, h_ref):
    dis = dis_ref[...]
    agg = a_ref[0] + a_ref[1] + h1_ref[...]
    z = jnp.maximum(dis * agg + b_ref[...], 0.0)
    h_ref[...] = dis * jnp.dot(z, w_ref[...],
                               preferred_element_type=jnp.float32)


def _k3(accp, h1p, disp, w2_big, b1rep):
    return pl.pallas_call(
        _k3_body,
        grid=(PR // _BP,),
        in_specs=[
            pl.BlockSpec((NC, _BP, 128), lambda i: (0, i, 0)),
            pl.BlockSpec((_BP, 128), lambda i: (i, 0)),
            pl.BlockSpec((_BP, 128), lambda i: (i, 0)),
            pl.BlockSpec((128, 128), lambda i: (0, 0)),
            pl.BlockSpec((1, 128), lambda i: (0, 0)),
        ],
        out_specs=pl.BlockSpec((_BP, 128), lambda i: (i, 0)),
        out_shape=jax.ShapeDtypeStruct((PR, 128), jnp.float32),
    )(accp, h1p, disp, w2_big, b1rep)


def _k5_body(a_ref, h2_ref, dis_ref, b_ref, o_ref):
    agg = a_ref[0] + a_ref[1] + h2_ref[...]
    o_ref[...] = dis_ref[...] * agg + b_ref[...]


def _k5(accp, h2p, disp, b2rep):
    return pl.pallas_call(
        _k5_body,
        grid=(PR // _BP,),
        in_specs=[
            pl.BlockSpec((NC, _BP, 128), lambda i: (0, i, 0)),
            pl.BlockSpec((_BP, 128), lambda i: (i, 0)),
            pl.BlockSpec((_BP, 128), lambda i: (i, 0)),
            pl.BlockSpec((1, 128), lambda i: (0, 0)),
        ],
        out_specs=pl.BlockSpec((_BP, 128), lambda i: (i, 0)),
        out_shape=jax.ShapeDtypeStruct((PR, 128), jnp.float32),
    )(accp, h2p, disp, b2rep)


def kernel(x, edge_index, W1, b1, W2, b2):
    ei4 = edge_index.reshape(2, NW, CH, CB)
    eye8 = jnp.eye(8, dtype=jnp.float32)
    w1_big = jnp.kron(eye8, W1)
    w2p = jnp.pad(W2, ((0, 0), (0, FW - F_OUT)))
    w2_big = jnp.kron(eye8, w2p)
    b1rep = jnp.tile(b1, 8).reshape(1, 128)
    b2rep = jnp.tile(jnp.pad(b2, (0, FW - F_OUT)), 8).reshape(1, 128)
    x_r = jnp.pad(x, ((0, NPAD - N), (0, 0))).reshape(PR, F_IN * 8)

    degp = _deg(ei4).reshape(NC, PR, 128)
    h1p, disp = _k1(x_r, w1_big, degp)

    acc1p = _agg(h1p.reshape(NPAD, FW), ei4).reshape(NC, PR, 128)
    h2p = _k3(acc1p, h1p, disp, w2_big, b1rep)
    acc2p = _agg(h2p.reshape(NPAD, FW), ei4).reshape(NC, PR, 128)
    outp = _k5(acc2p, h2p, disp, b2rep)
    return outp.reshape(NPAD, FW)[:N, :F_OUT]
